# baseline (speedup 1.0000x reference)
import functools

import jax
import jax.numpy as jnp
from jax import lax
from jax.experimental import pallas as pl
from jax.experimental.pallas import tpu as pltpu
from jax.experimental.pallas import tpu_sc as plsc

N = 10000
E = 160000
H = 256
HH = 128
NUM_LAYERS = 4
STEP = 0.1

NC = 2
NS = 16
K = 125
NCH = E // NS // K
NP = 10240
ROWS_PER_TILE = NP // NS
ZR = 64
KD = 125
DCH = E // (NC * NS) // KD
DW = 128

_MESH = plsc.VectorSubcoreMesh(core_axis_name="c", subcore_axis_name="s")


def _deg_body(col4, ones_hbm, zeros_hbm, degp, col_v, onesb, deg_s):
    cid = lax.axis_index("c")
    sid = lax.axis_index("s")

    @pl.when(sid == 0)
    def _():
        pltpu.sync_copy(zeros_hbm, deg_s)

    pltpu.sync_copy(col4.at[cid, sid], col_v)
    pltpu.sync_copy(ones_hbm, onesb)
    plsc.subcore_barrier()

    def body(j, carry):
        pltpu.sync_copy(onesb, deg_s.at[col_v.at[j]], add=True)
        return carry

    lax.fori_loop(0, DCH, body, 0)
    plsc.subcore_barrier()

    @pl.when(sid == 0)
    def _():
        pltpu.sync_copy(deg_s, degp.at[cid])


def _sc_deg(col4, ones16, zeros_np):
    f = pl.kernel(
        _deg_body,
        out_type=jax.ShapeDtypeStruct((NC, NP, DW), jnp.float32),
        mesh=_MESH,
        scratch_types=[
            pltpu.VMEM((DCH, KD), jnp.int32),
            pltpu.VMEM((KD, DW), jnp.float32),
            pltpu.VMEM_SHARED((NP, DW), jnp.float32),
        ],
    )
    return f(col4, ones16, zeros_np)


HCH = NCH // 2


def _agg_body(row4, col3, p2f, zeros_hbm, out, row_v, col_v, buf0, buf1,
              agg_s, sem0, sem1, ssem0, ssem1):
    cid = lax.axis_index("c")
    sid = lax.axis_index("s")

    @pl.when(sid == 0)
    def _():
        pltpu.sync_copy(zeros_hbm, agg_s)

    plsc.subcore_barrier()

    def gather_start(j, buf, sem):
        pltpu.make_async_copy(p2f.at[row_v.at[j]], buf, sem).start()

    def gather_wait(j, buf, sem):
        pltpu.make_async_copy(p2f.at[row_v.at[j]], buf, sem).wait()

    def scatter_start(j, buf, sem):
        pltpu.make_async_copy(buf, agg_s.at[col_v.at[j]], sem).start(add=True)

    def scatter_wait(j, buf, sem):
        pltpu.make_async_copy(buf, agg_s.at[col_v.at[j]], sem).wait()

    for half in range(2):
        pltpu.sync_copy(row4.at[cid, sid, pl.ds(half * HCH, HCH)], row_v)
        pltpu.sync_copy(col3.at[sid, pl.ds(half * HCH, HCH)], col_v)
        gather_start(0, buf0, sem0)
        gather_start(1, buf1, sem1)

        def body(jj, carry):
            j0 = 2 * jj
            gather_wait(j0, buf0, sem0)
            scatter_start(j0, buf0, ssem0)
            gather_wait(j0 + 1, buf1, sem1)
            scatter_start(j0 + 1, buf1, ssem1)
            scatter_wait(j0, buf0, ssem0)

            @pl.when(jj < HCH // 2 - 1)
            def _():
                gather_start(j0 + 2, buf0, sem0)

            scatter_wait(j0 + 1, buf1, ssem1)

            @pl.when(jj < HCH // 2 - 1)
            def _():
                gather_start(j0 + 3, buf1, sem1)

            return carry

        lax.fori_loop(0, HCH // 2, body, 0)

    plsc.subcore_barrier()

    @pl.when(sid == 0)
    def _():
        pltpu.sync_copy(agg_s, out.at[cid])


def _sc_layer(row4, col3, p2f, zeros128):
    f = pl.kernel(
        _agg_body,
        out_type=jax.ShapeDtypeStruct((NC, NP, HH), jnp.float32),
        mesh=_MESH,
        scratch_types=[
            pltpu.VMEM((HCH, K), jnp.int32),
            pltpu.VMEM((HCH, K), jnp.int32),
            pltpu.VMEM((K, HH), jnp.float32),
            pltpu.VMEM((K, HH), jnp.float32),
            pltpu.VMEM_SHARED((NP, HH), jnp.float32),
            pltpu.SemaphoreType.DMA,
            pltpu.SemaphoreType.DMA,
            pltpu.SemaphoreType.DMA,
            pltpu.SemaphoreType.DMA,
        ],
    )
    return f(row4, col3, p2f, zeros128)


BM = 1000


def _w_eff_block(W_pair):
    W0 = W_pair[:, :H]
    ri = lax.broadcasted_iota(jnp.int32, (H, H), 0)
    ci = lax.broadcasted_iota(jnp.int32, (H, H), 1)
    W0 = jnp.where(ci > ri, W0, 0.0)
    W0 = W0 + W0.T
    q = W_pair[:, H:H + 1]
    r = W_pair[:, H + 1:H + 2]
    wdiag = q * jnp.sum(jnp.abs(W0), axis=1, keepdims=True) + r
    return W0 + jnp.where(ci == ri, wdiag, 0.0)


def _prep_body(x, W_enc, W_pair, degp, h0, p2, W_eff, deg_inv):
    W = _w_eff_block(W_pair[...])
    W_eff[...] = W
    deg = degp[0, :, 0:1] + degp[1, :, 0:1]
    di = jnp.where(deg > 0, lax.rsqrt(deg), 0.0)
    deg_inv[...] = di
    h = lax.dot_general(x[...], W_enc[...], (((1,), (1,)), ((), ())),
                        preferred_element_type=jnp.float32)
    h0[...] = h
    p = di * jnp.dot(h, W, preferred_element_type=jnp.float32)
    p2[0] = p[:, :HH]
    p2[1] = p[:, HH:]


def _tc_prep(x, W_enc, W_pair, degp):
    grid = (N // BM,)
    return pl.pallas_call(
        _prep_body,
        grid=grid,
        in_specs=[
            pl.BlockSpec((BM, H), lambda i: (i, 0)),
            pl.BlockSpec((H, H), lambda i: (0, 0)),
            pl.BlockSpec((H, H + 2), lambda i: (0, 0)),
            pl.BlockSpec((NC, BM, DW), lambda i: (0, i, 0)),
        ],
        out_specs=[
            pl.BlockSpec((BM, H), lambda i: (i, 0)),
            pl.BlockSpec((NC, BM, HH), lambda i: (0, i, 0)),
            pl.BlockSpec((H, H), lambda i: (0, 0)),
            pl.BlockSpec((BM, 1), lambda i: (i, 0)),
        ],
        out_shape=[
            jax.ShapeDtypeStruct((N, H), jnp.float32),
            jax.ShapeDtypeStruct((NC, NP, HH), jnp.float32),
            jax.ShapeDtypeStruct((H, H), jnp.float32),
            jax.ShapeDtypeStruct((N, 1), jnp.float32),
        ],
    )(x, W_enc, W_pair, degp)


def _layer_math(h, x0, di, agg, ext_w, beta):
    aggf = jnp.concatenate([agg[0], agg[1]], axis=1)
    out = di * aggf - h * ext_w - x0 * beta
    return h + STEP * jnp.maximum(out, 0.0)


def _layer_body(h, x0, deg_inv, agg, ext_w, beta, W_eff, hn, p2):
    di = deg_inv[...]
    new_h = _layer_math(h[...], x0[...], di, agg[...], ext_w[...], beta[...])
    hn[...] = new_h
    p = di * jnp.dot(new_h, W_eff[...], preferred_element_type=jnp.float32)
    p2[0] = p[:, :HH]
    p2[1] = p[:, HH:]


def _final_body(h, x0, deg_inv, agg, ext_w, beta, hn):
    hn[...] = _layer_math(h[...], x0[...], deg_inv[...], agg[...], ext_w[...],
                          beta[...])


def _tc_layer(h, x0, deg_inv, agg, ext_w, beta, W_eff):
    grid = (N // BM,)
    return pl.pallas_call(
        _layer_body,
        grid=grid,
        in_specs=[
            pl.BlockSpec((BM, H), lambda i: (i, 0)),
            pl.BlockSpec((BM, H), lambda i: (i, 0)),
            pl.BlockSpec((BM, 1), lambda i: (i, 0)),
            pl.BlockSpec((NC, BM, HH), lambda i: (0, i, 0)),
            pl.BlockSpec((1, H), lambda i: (0, 0)),
            pl.BlockSpec((1, 1), lambda i: (0, 0)),
            pl.BlockSpec((H, H), lambda i: (0, 0)),
        ],
        out_specs=[
            pl.BlockSpec((BM, H), lambda i: (i, 0)),
            pl.BlockSpec((NC, BM, HH), lambda i: (0, i, 0)),
        ],
        out_shape=[
            jax.ShapeDtypeStruct((N, H), jnp.float32),
            jax.ShapeDtypeStruct((NC, NP, HH), jnp.float32),
        ],
    )(h, x0, deg_inv, agg, ext_w, beta, W_eff)


def _tc_final(h, x0, deg_inv, agg, ext_w, beta):
    grid = (N // BM,)
    return pl.pallas_call(
        _final_body,
        grid=grid,
        in_specs=[
            pl.BlockSpec((BM, H), lambda i: (i, 0)),
            pl.BlockSpec((BM, H), lambda i: (i, 0)),
            pl.BlockSpec((BM, 1), lambda i: (i, 0)),
            pl.BlockSpec((NC, BM, HH), lambda i: (0, i, 0)),
            pl.BlockSpec((1, H), lambda i: (0, 0)),
            pl.BlockSpec((1, 1), lambda i: (0, 0)),
        ],
        out_specs=pl.BlockSpec((BM, H), lambda i: (i, 0)),
        out_shape=jax.ShapeDtypeStruct((N, H), jnp.float32),
    )(h, x0, deg_inv, agg, ext_w, beta)


@jax.jit
def kernel(x, edge_index, W_enc, ext_w, beta, W_pair):
    row = edge_index[0].astype(jnp.int32)
    col = edge_index[1].astype(jnp.int32)
    row3 = row.reshape(NS, NCH, K)
    row4 = jnp.stack([row3, row3 + NP])
    col3 = col.reshape(NS, NCH, K)
    col4 = col.reshape(NC, NS, DCH, KD)
    ones16 = jnp.ones((KD, DW), jnp.float32)
    zeros16 = jnp.zeros((NP, DW), jnp.float32)
    zeros128 = jnp.zeros((NP, HH), jnp.float32)
    beta11 = beta.reshape(1, 1)

    degp = _sc_deg(col4, ones16, zeros16)
    h, p2, W_eff, deg_inv = _tc_prep(x, W_enc, W_pair, degp)
    x0 = h
    for l in range(NUM_LAYERS):
        agg = _sc_layer(row4, col3, p2.reshape(NC * NP, HH), zeros128)
        if l < NUM_LAYERS - 1:
            h, p2 = _tc_layer(h, x0, deg_inv, agg, ext_w, beta11, W_eff)
        else:
            h = _tc_final(h, x0, deg_inv, agg, ext_w, beta11)
    return h


# speedup vs baseline: 1.2387x; 1.2387x over previous
import functools

import jax
import jax.numpy as jnp
from jax import lax
from jax.experimental import pallas as pl
from jax.experimental.pallas import tpu as pltpu
from jax.experimental.pallas import tpu_sc as plsc

N = 10000
E = 160000
H = 256
HH = 128
NUM_LAYERS = 4
STEP = 0.1

NC = 2
NS = 16
K = 125
NCH = E // NS // K
NP = 10240
ROWS_PER_TILE = NP // NS
ZR = 64
KD = 125
DCH = E // (NC * NS) // KD
DW = 128

_MESH = plsc.VectorSubcoreMesh(core_axis_name="c", subcore_axis_name="s")


def _deg_body(col4, ones_hbm, zeros_hbm, degp, col_v, onesb, deg_s):
    cid = lax.axis_index("c")
    sid = lax.axis_index("s")

    @pl.when(sid == 0)
    def _():
        pltpu.sync_copy(zeros_hbm, deg_s)

    pltpu.sync_copy(col4.at[cid, sid], col_v)
    pltpu.sync_copy(ones_hbm, onesb)
    plsc.subcore_barrier()

    def body(j, carry):
        pltpu.sync_copy(onesb, deg_s.at[col_v.at[j]], add=True)
        return carry

    lax.fori_loop(0, DCH, body, 0)
    plsc.subcore_barrier()

    @pl.when(sid == 0)
    def _():
        pltpu.sync_copy(deg_s, degp.at[cid])


def _sc_deg(col4, ones16, zeros_np):
    f = pl.kernel(
        _deg_body,
        out_type=jax.ShapeDtypeStruct((NC, NP, DW), jnp.float32),
        mesh=_MESH,
        scratch_types=[
            pltpu.VMEM((DCH, KD), jnp.int32),
            pltpu.VMEM((KD, DW), jnp.float32),
            pltpu.VMEM_SHARED((NP, DW), jnp.float32),
        ],
    )
    return f(col4, ones16, zeros_np)


HCH = NCH // 2


def _agg_body(row4, col3, p2f, zeros_hbm, out, row_v, col_v, buf0, buf1,
              agg_s, sem0, sem1, ssem0, ssem1):
    cid = lax.axis_index("c")
    sid = lax.axis_index("s")

    @pl.when(sid == 0)
    def _():
        pltpu.sync_copy(zeros_hbm, agg_s)

    plsc.subcore_barrier()

    def gather_start(j, buf, sem):
        pltpu.make_async_copy(p2f.at[row_v.at[j]], buf, sem).start()

    def gather_wait(j, buf, sem):
        pltpu.make_async_copy(p2f.at[row_v.at[j]], buf, sem).wait()

    def scatter_start(j, buf, sem):
        pltpu.make_async_copy(buf, agg_s.at[col_v.at[j]], sem).start(add=True)

    def scatter_wait(j, buf, sem):
        pltpu.make_async_copy(buf, agg_s.at[col_v.at[j]], sem).wait()

    for half in range(2):
        pltpu.sync_copy(row4.at[cid, sid, pl.ds(half * HCH, HCH)], row_v)
        pltpu.sync_copy(col3.at[sid, pl.ds(half * HCH, HCH)], col_v)
        gather_start(0, buf0, sem0)
        gather_start(1, buf1, sem1)

        def body(jj, carry):
            j0 = 2 * jj
            gather_wait(j0, buf0, sem0)
            pltpu.sync_copy(buf0, agg_s.at[col_v.at[j0]], add=True)

            @pl.when(jj < HCH // 2 - 1)
            def _():
                gather_start(j0 + 2, buf0, sem0)

            gather_wait(j0 + 1, buf1, sem1)
            pltpu.sync_copy(buf1, agg_s.at[col_v.at[j0 + 1]], add=True)

            @pl.when(jj < HCH // 2 - 1)
            def _():
                gather_start(j0 + 3, buf1, sem1)

            return carry

        lax.fori_loop(0, HCH // 2, body, 0)

    plsc.subcore_barrier()

    @pl.when(sid == 0)
    def _():
        pltpu.sync_copy(agg_s, out.at[cid])


def _sc_layer(row4, col3, p2f, zeros128):
    f = pl.kernel(
        _agg_body,
        out_type=jax.ShapeDtypeStruct((NC, NP, HH), jnp.float32),
        mesh=_MESH,
        scratch_types=[
            pltpu.VMEM((HCH, K), jnp.int32),
            pltpu.VMEM((HCH, K), jnp.int32),
            pltpu.VMEM((K, HH), jnp.float32),
            pltpu.VMEM((K, HH), jnp.float32),
            pltpu.VMEM_SHARED((NP, HH), jnp.float32),
            pltpu.SemaphoreType.DMA,
            pltpu.SemaphoreType.DMA,
            pltpu.SemaphoreType.DMA,
            pltpu.SemaphoreType.DMA,
        ],
    )
    return f(row4, col3, p2f, zeros128)


BM = 1000


def _w_eff_block(W_pair):
    W0 = W_pair[:, :H]
    ri = lax.broadcasted_iota(jnp.int32, (H, H), 0)
    ci = lax.broadcasted_iota(jnp.int32, (H, H), 1)
    W0 = jnp.where(ci > ri, W0, 0.0)
    W0 = W0 + W0.T
    q = W_pair[:, H:H + 1]
    r = W_pair[:, H + 1:H + 2]
    wdiag = q * jnp.sum(jnp.abs(W0), axis=1, keepdims=True) + r
    return W0 + jnp.where(ci == ri, wdiag, 0.0)


def _prep_body(x, W_enc, W_pair, degp, h0, p2, W_eff, deg_inv):
    W = _w_eff_block(W_pair[...])
    W_eff[...] = W
    deg = degp[0, :, 0:1] + degp[1, :, 0:1]
    di = jnp.where(deg > 0, lax.rsqrt(deg), 0.0)
    deg_inv[...] = di
    h = lax.dot_general(x[...], W_enc[...], (((1,), (1,)), ((), ())),
                        preferred_element_type=jnp.float32)
    h0[...] = h
    p = di * jnp.dot(h, W, preferred_element_type=jnp.float32)
    p2[0] = p[:, :HH]
    p2[1] = p[:, HH:]


def _tc_prep(x, W_enc, W_pair, degp):
    grid = (N // BM,)
    return pl.pallas_call(
        _prep_body,
        grid=grid,
        in_specs=[
            pl.BlockSpec((BM, H), lambda i: (i, 0)),
            pl.BlockSpec((H, H), lambda i: (0, 0)),
            pl.BlockSpec((H, H + 2), lambda i: (0, 0)),
            pl.BlockSpec((NC, BM, DW), lambda i: (0, i, 0)),
        ],
        out_specs=[
            pl.BlockSpec((BM, H), lambda i: (i, 0)),
            pl.BlockSpec((NC, BM, HH), lambda i: (0, i, 0)),
            pl.BlockSpec((H, H), lambda i: (0, 0)),
            pl.BlockSpec((BM, 1), lambda i: (i, 0)),
        ],
        out_shape=[
            jax.ShapeDtypeStruct((N, H), jnp.float32),
            jax.ShapeDtypeStruct((NC, NP, HH), jnp.float32),
            jax.ShapeDtypeStruct((H, H), jnp.float32),
            jax.ShapeDtypeStruct((N, 1), jnp.float32),
        ],
    )(x, W_enc, W_pair, degp)


def _layer_math(h, x0, di, agg, ext_w, beta):
    aggf = jnp.concatenate([agg[0], agg[1]], axis=1)
    out = di * aggf - h * ext_w - x0 * beta
    return h + STEP * jnp.maximum(out, 0.0)


def _layer_body(h, x0, deg_inv, agg, ext_w, beta, W_eff, hn, p2):
    di = deg_inv[...]
    new_h = _layer_math(h[...], x0[...], di, agg[...], ext_w[...], beta[...])
    hn[...] = new_h
    p = di * jnp.dot(new_h, W_eff[...], preferred_element_type=jnp.float32)
    p2[0] = p[:, :HH]
    p2[1] = p[:, HH:]


def _final_body(h, x0, deg_inv, agg, ext_w, beta, hn):
    hn[...] = _layer_math(h[...], x0[...], deg_inv[...], agg[...], ext_w[...],
                          beta[...])


def _tc_layer(h, x0, deg_inv, agg, ext_w, beta, W_eff):
    grid = (N // BM,)
    return pl.pallas_call(
        _layer_body,
        grid=grid,
        in_specs=[
            pl.BlockSpec((BM, H), lambda i: (i, 0)),
            pl.BlockSpec((BM, H), lambda i: (i, 0)),
            pl.BlockSpec((BM, 1), lambda i: (i, 0)),
            pl.BlockSpec((NC, BM, HH), lambda i: (0, i, 0)),
            pl.BlockSpec((1, H), lambda i: (0, 0)),
            pl.BlockSpec((1, 1), lambda i: (0, 0)),
            pl.BlockSpec((H, H), lambda i: (0, 0)),
        ],
        out_specs=[
            pl.BlockSpec((BM, H), lambda i: (i, 0)),
            pl.BlockSpec((NC, BM, HH), lambda i: (0, i, 0)),
        ],
        out_shape=[
            jax.ShapeDtypeStruct((N, H), jnp.float32),
            jax.ShapeDtypeStruct((NC, NP, HH), jnp.float32),
        ],
    )(h, x0, deg_inv, agg, ext_w, beta, W_eff)


def _tc_final(h, x0, deg_inv, agg, ext_w, beta):
    grid = (N // BM,)
    return pl.pallas_call(
        _final_body,
        grid=grid,
        in_specs=[
            pl.BlockSpec((BM, H), lambda i: (i, 0)),
            pl.BlockSpec((BM, H), lambda i: (i, 0)),
            pl.BlockSpec((BM, 1), lambda i: (i, 0)),
            pl.BlockSpec((NC, BM, HH), lambda i: (0, i, 0)),
            pl.BlockSpec((1, H), lambda i: (0, 0)),
            pl.BlockSpec((1, 1), lambda i: (0, 0)),
        ],
        out_specs=pl.BlockSpec((BM, H), lambda i: (i, 0)),
        out_shape=jax.ShapeDtypeStruct((N, H), jnp.float32),
    )(h, x0, deg_inv, agg, ext_w, beta)


@jax.jit
def kernel(x, edge_index, W_enc, ext_w, beta, W_pair):
    row = edge_index[0].astype(jnp.int32)
    col = edge_index[1].astype(jnp.int32)
    row3 = row.reshape(NS, NCH, K)
    row4 = jnp.stack([row3, row3 + NP])
    col3 = col.reshape(NS, NCH, K)
    col4 = col.reshape(NC, NS, DCH, KD)
    ones16 = jnp.ones((KD, DW), jnp.float32)
    zeros16 = jnp.zeros((NP, DW), jnp.float32)
    zeros128 = jnp.zeros((NP, HH), jnp.float32)
    beta11 = beta.reshape(1, 1)

    degp = _sc_deg(col4, ones16, zeros16)
    h, p2, W_eff, deg_inv = _tc_prep(x, W_enc, W_pair, degp)
    x0 = h
    for l in range(NUM_LAYERS):
        agg = _sc_layer(row4, col3, p2.reshape(NC * NP, HH), zeros128)
        if l < NUM_LAYERS - 1:
            h, p2 = _tc_layer(h, x0, deg_inv, agg, ext_w, beta11, W_eff)
        else:
            h = _tc_final(h, x0, deg_inv, agg, ext_w, beta11)
    return h
